# Initial kernel scaffold; baseline (speedup 1.0000x reference)
import jax
import jax.numpy as jnp
from jax.experimental import pallas as pl


def kernel(center):
    raise NotImplementedError("write your pallas kernel here")


# speedup vs baseline: 36.4017x; 36.4017x over previous
import functools
import math

import jax
import jax.numpy as jnp
from jax.experimental import pallas as pl
from jax.experimental.pallas import tpu as pltpu

N = 4096
D = 64
BLK = 256
KNN = 10
BIG = 3.4e38


def _loss_kernel(xb_ref, c_ref, loss_ref, acc_ref):
    g = pl.program_id(0)
    xb = xb_ref[...]
    c = c_ref[...]

    sq_c = jnp.sum(c * c, axis=1)
    sq_r = jnp.sum(xb * xb, axis=1)
    dot = jax.lax.dot_general(
        xb, c, dimension_numbers=(((1,), (1,)), ((), ())),
        preferred_element_type=jnp.float32)
    d2 = sq_r[:, None] + sq_c[None, :] - 2.0 * dot
    d2 = jnp.maximum(d2, 0.0)

    row_ids = g * BLK + jax.lax.broadcasted_iota(jnp.int32, (BLK, N), 0)
    col_ids = jax.lax.broadcasted_iota(jnp.int32, (BLK, N), 1)
    d2 = jnp.where(row_ids == col_ids, BIG, d2)

    s = jnp.zeros((BLK,), jnp.float32)
    for _ in range(KNN):
        mn = jnp.min(d2, axis=1)
        s = s + jnp.sqrt(mn)
        d2 = jnp.where(d2 == mn[:, None], BIG, d2)
    m = s * (1.0 / KNN)

    bmax = jnp.max(m)
    bexp = jnp.sum(jnp.exp(m - bmax))
    bsum = jnp.sum(m)

    @pl.when(g == 0)
    def _():
        acc_ref[0] = bmax
        acc_ref[1] = bexp
        acc_ref[2] = bsum

    @pl.when(g > 0)
    def _():
        m0 = acc_ref[0]
        m1 = jnp.maximum(m0, bmax)
        acc_ref[1] = acc_ref[1] * jnp.exp(m0 - m1) + bexp * jnp.exp(bmax - m1)
        acc_ref[0] = m1
        acc_ref[2] = acc_ref[2] + bsum

    @pl.when(g == pl.num_programs(0) - 1)
    def _():
        lse = acc_ref[0] + jnp.log(acc_ref[1])
        loss_ref[0, 0] = lse - acc_ref[2] / N - math.log(N)


@functools.partial(jax.jit, static_argnames=())
def kernel(center):
    loss = pl.pallas_call(
        _loss_kernel,
        grid=(N // BLK,),
        in_specs=[
            pl.BlockSpec((BLK, D), lambda i: (i, 0)),
            pl.BlockSpec((N, D), lambda i: (0, 0)),
        ],
        out_specs=pl.BlockSpec((1, 1), lambda i: (0, 0),
                               memory_space=pltpu.SMEM),
        out_shape=jax.ShapeDtypeStruct((1, 1), jnp.float32),
        scratch_shapes=[pltpu.SMEM((3,), jnp.float32)],
    )(center, center)
    return loss.reshape(())


# speedup vs baseline: 72.4208x; 1.9895x over previous
import functools
import math

import jax
import jax.numpy as jnp
from jax.experimental import pallas as pl
from jax.experimental.pallas import tpu as pltpu

N = 4096
D = 64
BLK = 256
KNN = 10
CW = 128
NCH = N // CW
BIG = 3.4e38


def _loss_kernel(xb_ref, c_ref, loss_ref, acc_ref):
    g = pl.program_id(0)
    xb = xb_ref[...]
    c = c_ref[...]

    ones = jnp.ones((D,), jnp.float32)
    sq_c = jax.lax.dot_general(
        c * c, ones, dimension_numbers=(((1,), (0,)), ((), ())),
        preferred_element_type=jnp.float32)
    sq_r = jax.lax.dot_general(
        xb * xb, ones, dimension_numbers=(((1,), (0,)), ((), ())),
        preferred_element_type=jnp.float32)
    dot = jax.lax.dot_general(
        xb, c, dimension_numbers=(((1,), (1,)), ((), ())),
        preferred_element_type=jnp.float32)
    e = sq_r[:, None] + sq_c[None, :] - 2.0 * dot

    row_ids = g * BLK + jax.lax.broadcasted_iota(jnp.int32, (BLK, N), 0)
    col_ids = jax.lax.broadcasted_iota(jnp.int32, (BLK, N), 1)
    e = jnp.where(row_ids == col_ids, BIG, e)

    g1 = e[:, :CW]
    g2 = jnp.maximum(g1, e[:, CW:2 * CW])
    g1 = jnp.minimum(g1, e[:, CW:2 * CW])
    for k in range(2, NCH):
        e_k = e[:, k * CW:(k + 1) * CW]
        g2 = jnp.minimum(g2, jnp.maximum(g1, e_k))
        g1 = jnp.minimum(g1, e_k)

    s = jnp.zeros((BLK,), jnp.float32)
    for _ in range(KNN):
        v = jnp.min(g1, axis=1)
        s = s + jnp.sqrt(jnp.maximum(v, 0.0))
        hit = g1 == v[:, None]
        g1 = jnp.where(hit, g2, g1)
        g2 = jnp.where(hit, BIG, g2)
    m = s * (1.0 / KNN)

    bmax = jnp.max(m)
    bexp = jnp.sum(jnp.exp(m - bmax))
    bsum = jnp.sum(m)

    @pl.when(g == 0)
    def _():
        acc_ref[0] = bmax
        acc_ref[1] = bexp
        acc_ref[2] = bsum

    @pl.when(g > 0)
    def _():
        m0 = acc_ref[0]
        m1 = jnp.maximum(m0, bmax)
        acc_ref[1] = acc_ref[1] * jnp.exp(m0 - m1) + bexp * jnp.exp(bmax - m1)
        acc_ref[0] = m1
        acc_ref[2] = acc_ref[2] + bsum

    @pl.when(g == pl.num_programs(0) - 1)
    def _():
        lse = acc_ref[0] + jnp.log(acc_ref[1])
        loss_ref[0, 0] = lse - acc_ref[2] / N - math.log(N)


@functools.partial(jax.jit, static_argnames=())
def kernel(center):
    loss = pl.pallas_call(
        _loss_kernel,
        grid=(N // BLK,),
        in_specs=[
            pl.BlockSpec((BLK, D), lambda i: (i, 0)),
            pl.BlockSpec((N, D), lambda i: (0, 0)),
        ],
        out_specs=pl.BlockSpec((1, 1), lambda i: (0, 0),
                               memory_space=pltpu.SMEM),
        out_shape=jax.ShapeDtypeStruct((1, 1), jnp.float32),
        scratch_shapes=[pltpu.SMEM((3,), jnp.float32)],
    )(center, center)
    return loss.reshape(())


# speedup vs baseline: 96.0258x; 1.3259x over previous
import functools
import math

import jax
import jax.numpy as jnp
from jax.experimental import pallas as pl
from jax.experimental.pallas import tpu as pltpu

N = 4096
D = 64
BLK = 256
KNN = 10
CW = 128
NCH = N // CW
BIG = 3.4e38


def _loss_kernel(xb_ref, c_ref, loss_ref, caug_ref, acc_ref):
    g = pl.program_id(0)
    xb = xb_ref[...]

    ones = jnp.ones((D,), jnp.float32)

    @pl.when(g == 0)
    def _():
        c = c_ref[...]
        sq_c = jax.lax.dot_general(
            c * c, ones, dimension_numbers=(((1,), (0,)), ((), ())),
            preferred_element_type=jnp.float32)
        caug_ref[:, :D] = c * -2.0
        caug_ref[:, D] = sq_c
        caug_ref[:, D + 1] = jnp.ones((N,), jnp.float32)

    sq_r = jax.lax.dot_general(
        xb * xb, ones, dimension_numbers=(((1,), (0,)), ((), ())),
        preferred_element_type=jnp.float32)
    xb_aug = jnp.concatenate(
        [xb, jnp.ones((BLK, 1), jnp.float32), sq_r[:, None]], axis=1)
    e = jax.lax.dot_general(
        xb_aug, caug_ref[...], dimension_numbers=(((1,), (1,)), ((), ())),
        preferred_element_type=jnp.float32)

    row_ids = g * BLK + jax.lax.broadcasted_iota(jnp.int32, (BLK, N), 0)
    col_ids = jax.lax.broadcasted_iota(jnp.int32, (BLK, N), 1)
    e = jnp.where(row_ids == col_ids, BIG, e)

    g1 = e[:, :CW]
    g2 = jnp.maximum(g1, e[:, CW:2 * CW])
    g1 = jnp.minimum(g1, e[:, CW:2 * CW])
    for k in range(2, NCH):
        e_k = e[:, k * CW:(k + 1) * CW]
        g2 = jnp.minimum(g2, jnp.maximum(g1, e_k))
        g1 = jnp.minimum(g1, e_k)

    vs = []
    for _ in range(KNN):
        v = jnp.min(g1, axis=1)
        vs.append(v)
        hit = g1 == v[:, None]
        g1 = jnp.where(hit, g2, g1)
        g2 = jnp.where(hit, BIG, g2)
    vmat = jnp.stack(vs, axis=0)
    s = jnp.sum(jnp.sqrt(jnp.maximum(vmat, 0.0)), axis=0)
    m = s * (1.0 / KNN)

    bmax = jnp.max(m)
    bexp = jnp.sum(jnp.exp(m - bmax))
    bsum = jnp.sum(m)

    @pl.when(g == 0)
    def _():
        acc_ref[0] = bmax
        acc_ref[1] = bexp
        acc_ref[2] = bsum

    @pl.when(g > 0)
    def _():
        m0 = acc_ref[0]
        m1 = jnp.maximum(m0, bmax)
        acc_ref[1] = acc_ref[1] * jnp.exp(m0 - m1) + bexp * jnp.exp(bmax - m1)
        acc_ref[0] = m1
        acc_ref[2] = acc_ref[2] + bsum

    @pl.when(g == pl.num_programs(0) - 1)
    def _():
        lse = acc_ref[0] + jnp.log(acc_ref[1])
        loss_ref[0, 0] = lse - acc_ref[2] / N - math.log(N)


@functools.partial(jax.jit, static_argnames=())
def kernel(center):
    loss = pl.pallas_call(
        _loss_kernel,
        grid=(N // BLK,),
        in_specs=[
            pl.BlockSpec((BLK, D), lambda i: (i, 0)),
            pl.BlockSpec((N, D), lambda i: (0, 0)),
        ],
        out_specs=pl.BlockSpec((1, 1), lambda i: (0, 0),
                               memory_space=pltpu.SMEM),
        out_shape=jax.ShapeDtypeStruct((1, 1), jnp.float32),
        scratch_shapes=[pltpu.VMEM((N, D + 2), jnp.float32),
                        pltpu.SMEM((3,), jnp.float32)],
    )(center, center)
    return loss.reshape(())


# speedup vs baseline: 142.0447x; 1.4792x over previous
import functools
import math

import jax
import jax.numpy as jnp
from jax.experimental import pallas as pl
from jax.experimental.pallas import tpu as pltpu

N = 4096
D = 64
BLK = 512
KNN = 10
CW = 128
NCH = N // CW
BIG = 3.4e38


def _loss_kernel(xb_ref, c_ref, loss_ref, caug_ref, acc_ref):
    g = pl.program_id(0)
    xb = xb_ref[...]

    ones = jnp.ones((D,), jnp.float32)

    @pl.when(g == 0)
    def _():
        c = c_ref[...]
        sq_c = jax.lax.dot_general(
            c * c, ones, dimension_numbers=(((1,), (0,)), ((), ())),
            preferred_element_type=jnp.float32)
        caug_ref[:, :D] = c * -2.0
        caug_ref[:, D] = sq_c
        caug_ref[:, D + 1] = jnp.ones((N,), jnp.float32)

    sq_r = jax.lax.dot_general(
        xb * xb, ones, dimension_numbers=(((1,), (0,)), ((), ())),
        preferred_element_type=jnp.float32)
    xb_aug = jnp.concatenate(
        [xb, jnp.ones((BLK, 1), jnp.float32), sq_r[:, None]], axis=1)
    e = jax.lax.dot_general(
        xb_aug, caug_ref[...], dimension_numbers=(((1,), (1,)), ((), ())),
        preferred_element_type=jnp.float32)

    g1 = e[:, :CW]
    g2 = jnp.maximum(g1, e[:, CW:2 * CW])
    g1 = jnp.minimum(g1, e[:, CW:2 * CW])
    for k in range(2, NCH):
        e_k = e[:, k * CW:(k + 1) * CW]
        g2 = jnp.minimum(g2, jnp.maximum(g1, e_k))
        g1 = jnp.minimum(g1, e_k)

    vs = []
    for i in range(KNN + 1):
        v = jnp.min(g1, axis=1)
        if i > 0:
            vs.append(v)
        hit = g1 == v[:, None]
        g1 = jnp.where(hit, g2, g1)
        g2 = jnp.where(hit, BIG, g2)
    vmat = jnp.stack(vs, axis=0)
    s = jnp.sum(jnp.sqrt(jnp.maximum(vmat, 0.0)), axis=0)
    m = s * (1.0 / KNN)

    bmax = jnp.max(m)
    bexp = jnp.sum(jnp.exp(m - bmax))
    bsum = jnp.sum(m)

    @pl.when(g == 0)
    def _():
        acc_ref[0] = bmax
        acc_ref[1] = bexp
        acc_ref[2] = bsum

    @pl.when(g > 0)
    def _():
        m0 = acc_ref[0]
        m1 = jnp.maximum(m0, bmax)
        acc_ref[1] = acc_ref[1] * jnp.exp(m0 - m1) + bexp * jnp.exp(bmax - m1)
        acc_ref[0] = m1
        acc_ref[2] = acc_ref[2] + bsum

    @pl.when(g == pl.num_programs(0) - 1)
    def _():
        lse = acc_ref[0] + jnp.log(acc_ref[1])
        loss_ref[0, 0] = lse - acc_ref[2] / N - math.log(N)


@functools.partial(jax.jit, static_argnames=())
def kernel(center):
    loss = pl.pallas_call(
        _loss_kernel,
        grid=(N // BLK,),
        in_specs=[
            pl.BlockSpec((BLK, D), lambda i: (i, 0)),
            pl.BlockSpec((N, D), lambda i: (0, 0)),
        ],
        out_specs=pl.BlockSpec((1, 1), lambda i: (0, 0),
                               memory_space=pltpu.SMEM),
        out_shape=jax.ShapeDtypeStruct((1, 1), jnp.float32),
        scratch_shapes=[pltpu.VMEM((N, D + 2), jnp.float32),
                        pltpu.SMEM((3,), jnp.float32)],
    )(center, center)
    return loss.reshape(())


# speedup vs baseline: 166.2711x; 1.1706x over previous
import functools
import math

import jax
import jax.numpy as jnp
from jax.experimental import pallas as pl
from jax.experimental.pallas import tpu as pltpu

N = 4096
D = 64
BLK = 2048
KNN = 10
CW = 128
NCH = N // CW
BIG = 3.4e38


def _loss_kernel(xb_ref, c_ref, loss_ref, caug_ref, acc_ref):
    g = pl.program_id(0)
    xb = xb_ref[...]

    ones = jnp.ones((D,), jnp.float32)

    @pl.when(g == 0)
    def _():
        c = c_ref[...]
        sq_c = jax.lax.dot_general(
            c * c, ones, dimension_numbers=(((1,), (0,)), ((), ())),
            preferred_element_type=jnp.float32)
        caug_ref[:, :D] = c * -2.0
        caug_ref[:, D] = sq_c
        caug_ref[:, D + 1] = jnp.ones((N,), jnp.float32)

    sq_r = jax.lax.dot_general(
        xb * xb, ones, dimension_numbers=(((1,), (0,)), ((), ())),
        preferred_element_type=jnp.float32)
    xb_aug = jnp.concatenate(
        [xb, jnp.ones((BLK, 1), jnp.float32), sq_r[:, None]], axis=1)
    e = jax.lax.dot_general(
        xb_aug, caug_ref[...], dimension_numbers=(((1,), (1,)), ((), ())),
        preferred_element_type=jnp.float32)

    g1 = e[:, :CW]
    g2 = jnp.maximum(g1, e[:, CW:2 * CW])
    g1 = jnp.minimum(g1, e[:, CW:2 * CW])
    for k in range(2, NCH):
        e_k = e[:, k * CW:(k + 1) * CW]
        g2 = jnp.minimum(g2, jnp.maximum(g1, e_k))
        g1 = jnp.minimum(g1, e_k)

    vs = []
    for i in range(KNN + 1):
        v = jnp.min(g1, axis=1)
        if i > 0:
            vs.append(v)
        hit = g1 == v[:, None]
        g1 = jnp.where(hit, g2, g1)
        g2 = jnp.where(hit, BIG, g2)
    vmat = jnp.stack(vs, axis=0)
    s = jnp.sum(jnp.sqrt(jnp.maximum(vmat, 0.0)), axis=0)
    m = s * (1.0 / KNN)

    bmax = jnp.max(m)
    bexp = jnp.sum(jnp.exp(m - bmax))
    bsum = jnp.sum(m)

    @pl.when(g == 0)
    def _():
        acc_ref[0] = bmax
        acc_ref[1] = bexp
        acc_ref[2] = bsum

    @pl.when(g > 0)
    def _():
        m0 = acc_ref[0]
        m1 = jnp.maximum(m0, bmax)
        acc_ref[1] = acc_ref[1] * jnp.exp(m0 - m1) + bexp * jnp.exp(bmax - m1)
        acc_ref[0] = m1
        acc_ref[2] = acc_ref[2] + bsum

    @pl.when(g == pl.num_programs(0) - 1)
    def _():
        lse = acc_ref[0] + jnp.log(acc_ref[1])
        loss_ref[0, 0] = lse - acc_ref[2] / N - math.log(N)


@functools.partial(jax.jit, static_argnames=())
def kernel(center):
    loss = pl.pallas_call(
        _loss_kernel,
        grid=(N // BLK,),
        in_specs=[
            pl.BlockSpec((BLK, D), lambda i: (i, 0)),
            pl.BlockSpec((N, D), lambda i: (0, 0)),
        ],
        out_specs=pl.BlockSpec((1, 1), lambda i: (0, 0),
                               memory_space=pltpu.SMEM),
        out_shape=jax.ShapeDtypeStruct((1, 1), jnp.float32),
        scratch_shapes=[pltpu.VMEM((N, D + 2), jnp.float32),
                        pltpu.SMEM((3,), jnp.float32)],
    )(center, center)
    return loss.reshape(())


# speedup vs baseline: 170.0527x; 1.0227x over previous
import functools
import math

import jax
import jax.numpy as jnp
from jax.experimental import pallas as pl
from jax.experimental.pallas import tpu as pltpu

N = 4096
D = 64
BLK = 2048
KNN = 10
CW = 128
NCH = N // CW
BIG = 3.4e38


def _loss_kernel(c_ref, loss_ref, caug_ref, acc_ref):
    g = pl.program_id(0)
    xb = c_ref[pl.ds(g * BLK, BLK), :]

    ones = jnp.ones((D,), jnp.float32)

    @pl.when(g == 0)
    def _():
        c = c_ref[...]
        sq_c = jax.lax.dot_general(
            c * c, ones, dimension_numbers=(((1,), (0,)), ((), ())),
            preferred_element_type=jnp.float32)
        caug_ref[:, :D] = c * -2.0
        caug_ref[:, D] = sq_c
        caug_ref[:, D + 1] = jnp.ones((N,), jnp.float32)

    sq_r = jax.lax.dot_general(
        xb * xb, ones, dimension_numbers=(((1,), (0,)), ((), ())),
        preferred_element_type=jnp.float32)
    xb_aug = jnp.concatenate(
        [xb, jnp.ones((BLK, 1), jnp.float32), sq_r[:, None]], axis=1)
    e = jax.lax.dot_general(
        xb_aug, caug_ref[...], dimension_numbers=(((1,), (1,)), ((), ())),
        preferred_element_type=jnp.float32)

    g1 = e[:, :CW]
    g2 = jnp.maximum(g1, e[:, CW:2 * CW])
    g1 = jnp.minimum(g1, e[:, CW:2 * CW])
    for k in range(2, NCH):
        e_k = e[:, k * CW:(k + 1) * CW]
        g2 = jnp.minimum(g2, jnp.maximum(g1, e_k))
        g1 = jnp.minimum(g1, e_k)

    vs = []
    for i in range(KNN + 1):
        v = jnp.min(g1, axis=1)
        if i > 0:
            vs.append(v)
        hit = g1 == v[:, None]
        g1 = jnp.where(hit, g2, g1)
        g2 = jnp.where(hit, BIG, g2)
    vmat = jnp.stack(vs, axis=0)
    s = jnp.sum(jnp.sqrt(jnp.maximum(vmat, 0.0)), axis=0)
    m = s * (1.0 / KNN)

    bmax = jnp.max(m)
    bexp = jnp.sum(jnp.exp(m - bmax))
    bsum = jnp.sum(m)

    @pl.when(g == 0)
    def _():
        acc_ref[0] = bmax
        acc_ref[1] = bexp
        acc_ref[2] = bsum

    @pl.when(g > 0)
    def _():
        m0 = acc_ref[0]
        m1 = jnp.maximum(m0, bmax)
        acc_ref[1] = acc_ref[1] * jnp.exp(m0 - m1) + bexp * jnp.exp(bmax - m1)
        acc_ref[0] = m1
        acc_ref[2] = acc_ref[2] + bsum

    @pl.when(g == pl.num_programs(0) - 1)
    def _():
        lse = acc_ref[0] + jnp.log(acc_ref[1])
        loss_ref[0, 0] = lse - acc_ref[2] / N - math.log(N)


@functools.partial(jax.jit, static_argnames=())
def kernel(center):
    loss = pl.pallas_call(
        _loss_kernel,
        grid=(N // BLK,),
        in_specs=[
            pl.BlockSpec((N, D), lambda i: (0, 0)),
        ],
        out_specs=pl.BlockSpec((1, 1), lambda i: (0, 0),
                               memory_space=pltpu.SMEM),
        out_shape=jax.ShapeDtypeStruct((1, 1), jnp.float32),
        scratch_shapes=[pltpu.VMEM((N, D + 2), jnp.float32),
                        pltpu.SMEM((3,), jnp.float32)],
    )(center)
    return loss.reshape(())
